# Initial kernel scaffold; baseline (speedup 1.0000x reference)
import jax
import jax.numpy as jnp
from jax.experimental import pallas as pl


def kernel(z_src, z_dst, edge_label_index):
    raise NotImplementedError("write your pallas kernel here")


# speedup vs baseline: 2.3451x; 2.3451x over previous
import functools

import jax
import jax.numpy as jnp
from jax import lax
from jax.experimental import pallas as pl
from jax.experimental.pallas import tpu as pltpu
from jax.experimental.pallas import tpu_sc as plsc

NC = 2
NS = 16
NW = NC * NS
L = 16
CHUNK = 64


@functools.partial(jax.jit, static_argnames=("epw", "d"))
def _decode(src_idx, dst_idx, z_src, z_dst, *, epw, d):
    e_pad = src_idx.shape[0]
    n_chunks = epw // CHUNK
    mesh = plsc.VectorSubcoreMesh(
        core_axis_name="c", subcore_axis_name="s", num_cores=NC,
        num_subcores=NS)

    @functools.partial(
        pl.kernel,
        out_type=jax.ShapeDtypeStruct((e_pad,), jnp.float32),
        mesh=mesh,
        compiler_params=pltpu.CompilerParams(needs_layout_passes=False),
        scratch_types=[
            pltpu.VMEM((epw,), jnp.int32),
            pltpu.VMEM((epw,), jnp.int32),
            pltpu.VMEM((CHUNK, d), jnp.float32),
            pltpu.VMEM((CHUNK, d), jnp.float32),
            pltpu.VMEM((L * L,), jnp.float32),
            pltpu.VMEM((CHUNK,), jnp.float32),
            pltpu.SemaphoreType.DMA,
            pltpu.SemaphoreType.DMA,
        ],
    )
    def sc_decode(src_hbm, dst_hbm, zsrc_hbm, zdst_hbm, out_hbm,
                  sidx_v, didx_v, srows, drows, tbuf, outv, sem_s, sem_d):
        wid = lax.axis_index("s") * NC + lax.axis_index("c")
        base = wid * epw
        pltpu.sync_copy(src_hbm.at[pl.ds(base, epw)], sidx_v)
        pltpu.sync_copy(dst_hbm.at[pl.ds(base, epw)], didx_v)
        lanes16 = lax.iota(jnp.int32, L) * L

        def chunk_body(ci, carry):
            cb = ci * CHUNK
            cps = pltpu.async_copy(
                zsrc_hbm.at[sidx_v.at[pl.ds(cb, CHUNK)]], srows, sem_s)
            cpd = pltpu.async_copy(
                zdst_hbm.at[didx_v.at[pl.ds(cb, CHUNK)]], drows, sem_d)
            cps.wait()
            cpd.wait()

            def group_body(g, carry2):
                gb = g * L
                for e16 in range(L):
                    e = gb + e16
                    acc = srows[e, pl.ds(0, L)] * drows[e, pl.ds(0, L)]
                    for j in range(1, d // L):
                        acc = acc + (srows[e, pl.ds(j * L, L)]
                                     * drows[e, pl.ds(j * L, L)])
                    plsc.store_scatter(tbuf, [lanes16 + e16], acc)
                dots = tbuf[pl.ds(0, L)]
                for l in range(1, L):
                    dots = dots + tbuf[pl.ds(l * L, L)]
                outv[pl.ds(gb, L)] = dots
                return carry2

            lax.fori_loop(0, CHUNK // L, group_body, 0)
            pltpu.sync_copy(outv, out_hbm.at[pl.ds(base + cb, CHUNK)])
            return carry

        lax.fori_loop(0, n_chunks, chunk_body, 0)

    return sc_decode(src_idx, dst_idx, z_src, z_dst)


def kernel(z_src, z_dst, edge_label_index):
    src = edge_label_index[0].astype(jnp.int32)
    dst = edge_label_index[1].astype(jnp.int32)
    e = src.shape[0]
    d = z_src.shape[1]
    grain = NW * CHUNK
    e_pad = -(-e // grain) * grain
    if e_pad != e:
        src = jnp.concatenate([src, jnp.zeros((e_pad - e,), jnp.int32)])
        dst = jnp.concatenate([dst, jnp.zeros((e_pad - e,), jnp.int32)])
    out = _decode(src, dst, z_src, z_dst, epw=e_pad // NW, d=d)
    return out[:e]
